# Initial kernel scaffold; baseline (speedup 1.0000x reference)
import jax
import jax.numpy as jnp
from jax.experimental import pallas as pl


def kernel(hidden_states, gate_weight, gate_up_weights, down_weights, shared_gate_w, shared_up_w, shared_down_w):
    raise NotImplementedError("write your pallas kernel here")


# speedup vs baseline: 1.1442x; 1.1442x over previous
import functools

import jax
import jax.numpy as jnp
from jax.experimental import pallas as pl
from jax.experimental.pallas import tpu as pltpu

B, S, D = 1, 2048, 1024
E, K = 16, 2
DFF = 704
DFF_SH = 1408

TT = 256
NTT = S // TT


def _moe_kernel(x_ref, gw_ref, guw_ref, dw_ref, sgw_ref, suw_ref, sdw_ref,
                y_ref, scores_ref):
    e = pl.program_id(1)

    x = x_ref[...]

    @pl.when(e == 0)
    def _():
        logits = jnp.dot(x, gw_ref[...].T, preferred_element_type=jnp.float32)
        m = jnp.max(logits, axis=-1, keepdims=True)
        ex = jnp.exp(logits - m)
        scores_ref[...] = ex / jnp.sum(ex, axis=-1, keepdims=True)

    scores = scores_ref[...]
    iota = jax.lax.broadcasted_iota(jnp.int32, scores.shape, 1)
    v1 = jnp.max(scores, axis=-1, keepdims=True)
    i1 = jnp.min(jnp.where(scores == v1, iota, E), axis=-1, keepdims=True)
    masked = jnp.where(iota == i1, -jnp.inf, scores)
    v2 = jnp.max(masked, axis=-1, keepdims=True)
    i2 = jnp.min(jnp.where(masked == v2, iota, E), axis=-1, keepdims=True)
    w_e = v1[:, 0] * (i1[:, 0] == e) + v2[:, 0] * (i2[:, 0] == e)

    wg = guw_ref[0, :DFF, :]
    wu = guw_ref[0, DFF:, :]
    g = jnp.dot(x, wg.T, preferred_element_type=jnp.float32)
    u = jnp.dot(x, wu.T, preferred_element_type=jnp.float32)
    h = (g * jax.nn.sigmoid(g)) * u
    out = jnp.dot(h, dw_ref[0].T, preferred_element_type=jnp.float32)
    contrib = out * w_e[:, None]

    @pl.when(e == 0)
    def _():
        y_ref[...] = contrib

    @pl.when(e != 0)
    def _():
        y_ref[...] += contrib

    @pl.when(e == E - 1)
    def _():
        sg = jnp.dot(x, sgw_ref[...].T, preferred_element_type=jnp.float32)
        su = jnp.dot(x, suw_ref[...].T, preferred_element_type=jnp.float32)
        hs = (sg * jax.nn.sigmoid(sg)) * su
        y_ref[...] += jnp.dot(hs, sdw_ref[...].T,
                              preferred_element_type=jnp.float32)


@jax.jit
def kernel(hidden_states, gate_weight, gate_up_weights, down_weights,
           shared_gate_w, shared_up_w, shared_down_w):
    x = hidden_states.reshape(-1, D)

    y = pl.pallas_call(
        _moe_kernel,
        grid=(NTT, E),
        in_specs=[
            pl.BlockSpec((TT, D), lambda t, e: (t, 0)),
            pl.BlockSpec((E, D), lambda t, e: (0, 0)),
            pl.BlockSpec((1, 2 * DFF, D), lambda t, e: (e, 0, 0)),
            pl.BlockSpec((1, D, DFF), lambda t, e: (e, 0, 0)),
            pl.BlockSpec((DFF_SH, D), lambda t, e: (0, 0)),
            pl.BlockSpec((DFF_SH, D), lambda t, e: (0, 0)),
            pl.BlockSpec((D, DFF_SH), lambda t, e: (0, 0)),
        ],
        out_specs=pl.BlockSpec((TT, D), lambda t, e: (t, 0)),
        out_shape=jax.ShapeDtypeStruct((S, D), jnp.float32),
        scratch_shapes=[pltpu.VMEM((TT, E), jnp.float32)],
    )(x, gate_weight, gate_up_weights, down_weights,
      shared_gate_w, shared_up_w, shared_down_w)

    return y.reshape(B, S, D)


# speedup vs baseline: 1.4441x; 1.2621x over previous
import functools

import jax
import jax.numpy as jnp
from jax.experimental import pallas as pl
from jax.experimental.pallas import tpu as pltpu

B, S, D = 1, 2048, 1024
E, K = 16, 2
DFF = 704
DFF_SH = 1408

TT = 512
NTT = S // TT


def _moe_kernel(x_ref, xb_ref, gw_ref, guw_ref, dw_ref, sgw_ref, suw_ref,
                sdw_ref, y_ref, scores_ref):
    e = pl.program_id(1)

    xb = xb_ref[...]

    @pl.when(e == 0)
    def _():
        x = x_ref[...]
        logits = jnp.dot(x, gw_ref[...].T, preferred_element_type=jnp.float32)
        m = jnp.max(logits, axis=-1, keepdims=True)
        ex = jnp.exp(logits - m)
        scores_ref[...] = ex / jnp.sum(ex, axis=-1, keepdims=True)

    scores = scores_ref[...]
    iota = jax.lax.broadcasted_iota(jnp.int32, scores.shape, 1)
    v1 = jnp.max(scores, axis=-1, keepdims=True)
    i1 = jnp.min(jnp.where(scores == v1, iota, E), axis=-1, keepdims=True)
    masked = jnp.where(iota == i1, -jnp.inf, scores)
    v2 = jnp.max(masked, axis=-1, keepdims=True)
    i2 = jnp.min(jnp.where(masked == v2, iota, E), axis=-1, keepdims=True)
    w_e = v1[:, 0] * (i1[:, 0] == e) + v2[:, 0] * (i2[:, 0] == e)

    gu = jnp.dot(xb, guw_ref[0].T, preferred_element_type=jnp.float32)
    g = gu[:, :DFF]
    u = gu[:, DFF:]
    h = (g * jax.nn.sigmoid(g)) * u
    out = jnp.dot(h.astype(jnp.bfloat16), dw_ref[0].T,
                  preferred_element_type=jnp.float32)
    contrib = out * w_e[:, None]

    @pl.when(e == 0)
    def _():
        y_ref[...] = contrib

    @pl.when(e != 0)
    def _():
        y_ref[...] += contrib

    @pl.when(e == E - 1)
    def _():
        su = jnp.dot(xb, jnp.concatenate([sgw_ref[...], suw_ref[...]], 0).T,
                     preferred_element_type=jnp.float32)
        sg = su[:, :DFF_SH]
        sup = su[:, DFF_SH:]
        hs = (sg * jax.nn.sigmoid(sg)) * sup
        y_ref[...] += jnp.dot(hs.astype(jnp.bfloat16), sdw_ref[...].T,
                              preferred_element_type=jnp.float32)


@jax.jit
def kernel(hidden_states, gate_weight, gate_up_weights, down_weights,
           shared_gate_w, shared_up_w, shared_down_w):
    x = hidden_states.reshape(-1, D)
    xb = x.astype(jnp.bfloat16)
    guw = gate_up_weights.astype(jnp.bfloat16)
    dw = down_weights.astype(jnp.bfloat16)
    sgw = shared_gate_w.astype(jnp.bfloat16)
    suw = shared_up_w.astype(jnp.bfloat16)
    sdw = shared_down_w.astype(jnp.bfloat16)

    y = pl.pallas_call(
        _moe_kernel,
        grid=(NTT, E),
        in_specs=[
            pl.BlockSpec((TT, D), lambda t, e: (t, 0)),
            pl.BlockSpec((TT, D), lambda t, e: (t, 0)),
            pl.BlockSpec((E, D), lambda t, e: (0, 0)),
            pl.BlockSpec((1, 2 * DFF, D), lambda t, e: (e, 0, 0)),
            pl.BlockSpec((1, D, DFF), lambda t, e: (e, 0, 0)),
            pl.BlockSpec((DFF_SH, D), lambda t, e: (0, 0)),
            pl.BlockSpec((DFF_SH, D), lambda t, e: (0, 0)),
            pl.BlockSpec((D, DFF_SH), lambda t, e: (0, 0)),
        ],
        out_specs=pl.BlockSpec((TT, D), lambda t, e: (t, 0)),
        out_shape=jax.ShapeDtypeStruct((S, D), jnp.float32),
        scratch_shapes=[pltpu.VMEM((TT, E), jnp.float32)],
    )(x, xb, gate_weight, guw, dw, sgw, suw, sdw)

    return y.reshape(B, S, D)
